# Initial kernel scaffold; baseline (speedup 1.0000x reference)
import jax
import jax.numpy as jnp
from jax.experimental import pallas as pl


def kernel(x, edge_index, edge_attr, batch, x_id, mask, num_nodes, num_graphs, node_emb_table, edge_emb_table, W0, b0, W1, b1, W2, b2):
    raise NotImplementedError("write your pallas kernel here")


# speedup vs baseline: 4.1662x; 4.1662x over previous
import functools

import jax
import jax.numpy as jnp
from jax import lax
from jax.experimental import pallas as pl
from jax.experimental.pallas import tpu as pltpu
from jax.experimental.pallas import tpu_sc as plsc

F32 = jnp.float32
N = 10000
E = 320000
G = 16
D = 128
NREL = 64

NC, NS = 2, 16
NW = NC * NS
EPW = E // NW
K = 80
NB = EPW // K
NP = 10240
RPT = NP // NS

_MESH = plsc.VectorSubcoreMesh(core_axis_name="c", subcore_axis_name="s")


@functools.partial(
    pl.kernel,
    out_type=jax.ShapeDtypeStruct((NC * NP, D), F32),
    mesh=_MESH,
    scratch_types=[
        pltpu.VMEM((K,), jnp.int32),
        pltpu.VMEM((K,), jnp.int32),
        pltpu.VMEM((K, D), F32),
        pltpu.SemaphoreType.DMA,
        pltpu.VMEM_SHARED((NP, D), F32),
    ],
)
def _sc_pass0(taug_h, col_h, attr_h, z_h, nr_o, attrb, colb, gb, sem, acc):
    cid = lax.axis_index("c")
    sub = lax.axis_index("s")
    wid = sub * NC + cid
    pltpu.sync_copy(z_h.at[pl.ds(sub * RPT, RPT)], acc.at[pl.ds(sub * RPT, RPT)])
    plsc.subcore_barrier()

    def blk(b, carry):
        base = wid * EPW + b * K
        pltpu.sync_copy(attr_h.at[pl.ds(base, K)], attrb)
        pltpu.sync_copy(col_h.at[pl.ds(base, K)], colb)
        pltpu.async_copy(taug_h.at[attrb], gb, sem).wait()
        pltpu.sync_copy(gb, acc.at[colb], add=True)
        return carry

    lax.fori_loop(0, NB, blk, 0)
    plsc.subcore_barrier()
    pltpu.sync_copy(acc.at[pl.ds(sub * RPT, RPT)],
                    nr_o.at[pl.ds(cid * NP + sub * RPT, RPT)])


@functools.partial(
    pl.kernel,
    out_type=jax.ShapeDtypeStruct((NC * NP, D), F32),
    mesh=_MESH,
    scratch_types=[
        pltpu.VMEM((K,), jnp.int32),
        pltpu.VMEM((K, D), F32),
        pltpu.VMEM_SHARED((NP, D), F32),
    ],
)
def _sc_deg(col_h, ones_h, z_h, deg_o, colb, vb, acc):
    cid = lax.axis_index("c")
    sub = lax.axis_index("s")
    wid = sub * NC + cid
    pltpu.sync_copy(z_h.at[pl.ds(sub * RPT, RPT)], acc.at[pl.ds(sub * RPT, RPT)])
    pltpu.sync_copy(ones_h, vb)
    plsc.subcore_barrier()

    def blk(b, carry):
        base = wid * EPW + b * K
        pltpu.sync_copy(col_h.at[pl.ds(base, K)], colb)
        pltpu.sync_copy(vb, acc.at[colb], add=True)
        return carry

    lax.fori_loop(0, NB, blk, 0)
    plsc.subcore_barrier()
    pltpu.sync_copy(acc.at[pl.ds(sub * RPT, RPT)],
                    deg_o.at[pl.ds(cid * NP + sub * RPT, RPT)])


def _make_edge_pass(gather_third: bool, write_ea: bool):
    if write_ea:
        out_type = [jax.ShapeDtypeStruct((E, D), F32),
                    jax.ShapeDtypeStruct((NC * NP, D), F32)]
    else:
        out_type = jax.ShapeDtypeStruct((NC * NP, D), F32)

    def body(p_h, q_h, s_h, row_h, col_h, attr_h, z_h, *rest):
        if write_ea:
            ea_o, nr_o = rest[0], rest[1]
            scr = rest[2:]
        else:
            nr_o = rest[0]
            scr = rest[1:]
        rowb, colb, attrb, pb, qb, mb, vb, s0, s1, s2, acc = scr
        cid = lax.axis_index("c")
        sub = lax.axis_index("s")
        wid = sub * NC + cid
        pltpu.sync_copy(z_h.at[pl.ds(sub * RPT, RPT)],
                        acc.at[pl.ds(sub * RPT, RPT)])
        plsc.subcore_barrier()

        def blk(b, carry):
            base = wid * EPW + b * K
            pltpu.sync_copy(row_h.at[pl.ds(base, K)], rowb)
            pltpu.sync_copy(col_h.at[pl.ds(base, K)], colb)
            cp_p = pltpu.async_copy(p_h.at[rowb], pb, s0)
            cp_q = pltpu.async_copy(q_h.at[colb], qb, s1)
            if gather_third:
                pltpu.sync_copy(attr_h.at[pl.ds(base, K)], attrb)
                cp_s = pltpu.async_copy(s_h.at[attrb], mb, s2)
            else:
                cp_s = pltpu.async_copy(s_h.at[pl.ds(base, K)], mb, s2)
            cp_p.wait()
            cp_q.wait()
            cp_s.wait()

            def chunk(i, c2):
                for j in range(D // 16):
                    sl = pl.ds(j * 16, 16)
                    vb[i, sl] = jnp.maximum(pb[i, sl] + qb[i, sl] + mb[i, sl],
                                            0.0)
                return c2

            lax.fori_loop(0, K, chunk, 0)
            if write_ea:
                pltpu.sync_copy(vb, ea_o.at[pl.ds(base, K)])
            pltpu.sync_copy(vb, acc.at[colb], add=True)
            return carry

        lax.fori_loop(0, NB, blk, 0)
        plsc.subcore_barrier()
        pltpu.sync_copy(acc.at[pl.ds(sub * RPT, RPT)],
                        nr_o.at[pl.ds(cid * NP + sub * RPT, RPT)])

    return pl.kernel(
        body,
        out_type=out_type,
        mesh=_MESH,
        scratch_types=[
            pltpu.VMEM((K,), jnp.int32),
            pltpu.VMEM((K,), jnp.int32),
            pltpu.VMEM((K,), jnp.int32),
            pltpu.VMEM((K, D), F32),
            pltpu.VMEM((K, D), F32),
            pltpu.VMEM((K, D), F32),
            pltpu.VMEM((K, D), F32),
            pltpu.SemaphoreType.DMA,
            pltpu.SemaphoreType.DMA,
            pltpu.SemaphoreType.DMA,
            pltpu.VMEM_SHARED((NP, D), F32),
        ],
    )


_sc_edge_first = _make_edge_pass(gather_third=True, write_ea=True)
_sc_edge_mid = _make_edge_pass(gather_third=False, write_ea=True)
_sc_edge_last = _make_edge_pass(gather_third=False, write_ea=False)


def _tc_c0(table, wct):
    def body(t_ref, w_ref, o_ref):
        o_ref[...] = jnp.dot(t_ref[...], w_ref[...],
                             preferred_element_type=F32)

    return pl.pallas_call(
        body, out_shape=jax.ShapeDtypeStruct((NREL, D), F32))(table, wct)


def _tc_rden(dega, degb):
    def body(a_ref, b_ref, o_ref):
        o_ref[...] = 1.0 / (a_ref[...] + b_ref[...] + 1.0)

    return pl.pallas_call(
        body, out_shape=jax.ShapeDtypeStruct((N, 1), F32))(dega, degb)


_PQ_BLK = 2000


def _tc_pq(acc_a, acc_b, rden, xz, wat, wxat, wbt, wxbt, bias):
    def body(a_ref, b_ref, r_ref, xz_ref, wat_ref, wxat_ref, wbt_ref,
             wxbt_ref, bias_ref, p_ref, q_ref):
        nr = (a_ref[...] + b_ref[...]) * r_ref[...]
        xzb = xz_ref[...]
        p_ref[...] = (jnp.dot(nr, wat_ref[...], preferred_element_type=F32)
                      + jnp.dot(xzb, wxat_ref[...], preferred_element_type=F32)
                      + bias_ref[...])
        q_ref[...] = (jnp.dot(nr, wbt_ref[...], preferred_element_type=F32)
                      + jnp.dot(xzb, wxbt_ref[...], preferred_element_type=F32))

    grid = (N // _PQ_BLK,)
    row_spec = lambda shp: pl.BlockSpec((_PQ_BLK, shp), lambda i: (i, 0))
    full = lambda a, b: pl.BlockSpec((a, b), lambda i: (0, 0))
    return pl.pallas_call(
        body,
        grid=grid,
        in_specs=[row_spec(D), row_spec(D), row_spec(1), row_spec(6),
                  full(D, D), full(6, D), full(D, D), full(6, D), full(1, D)],
        out_specs=[pl.BlockSpec((_PQ_BLK, D), lambda i: (i, 0))] * 2,
        out_shape=[jax.ShapeDtypeStruct((N, D), F32)] * 2,
    )(acc_a, acc_b, rden, xz, wat, wxat, wbt, wxbt, bias)


_M_BLK = 8000


def _tc_m(ea, wct):
    def body(e_ref, w_ref, o_ref):
        o_ref[...] = jnp.dot(e_ref[...], w_ref[...],
                             preferred_element_type=F32)

    return pl.pallas_call(
        body,
        grid=(E // _M_BLK,),
        in_specs=[pl.BlockSpec((_M_BLK, D), lambda i: (i, 0)),
                  pl.BlockSpec((D, D), lambda i: (0, 0))],
        out_specs=pl.BlockSpec((_M_BLK, D), lambda i: (i, 0)),
        out_shape=jax.ShapeDtypeStruct((E, D), F32),
    )(ea, wct)


_F_BLK = 1000


def _tc_final(acc_a, acc_b, rden, batch2, head2, tail2):
    def body(a_ref, b_ref, r_ref, bat_ref, h_ref, t_ref, o_ref):
        step = pl.program_id(0)
        nr = (a_ref[...] + b_ref[...]) * r_ref[...]
        bat = bat_ref[...]
        gidx = (step * _F_BLK
                + lax.broadcasted_iota(jnp.int32, (_F_BLK, 1), 0))

        @pl.when(step == 0)
        def _():
            o_ref[...] = jnp.concatenate(
                [jnp.full((G, D), -jnp.inf, F32), jnp.zeros((G, 2 * D), F32)],
                axis=1)

        parts = []
        for g in range(G):
            m = jnp.where(bat == g, nr, -jnp.inf)
            parts.append(jnp.max(m, axis=0, keepdims=True))
        mx = jnp.concatenate(parts, axis=0)
        o_ref[:, 0:D] = jnp.maximum(o_ref[:, 0:D], mx)

        oh_h = (gidx == h_ref[...]).astype(F32)
        oh_t = (gidx == t_ref[...]).astype(F32)
        hp = lax.dot_general(oh_h, nr, (((0,), (0,)), ((), ())),
                             preferred_element_type=F32)
        tp = lax.dot_general(oh_t, nr, (((0,), (0,)), ((), ())),
                             preferred_element_type=F32)
        o_ref[:, D:2 * D] += hp
        o_ref[:, 2 * D:3 * D] += tp

    row_spec = lambda shp: pl.BlockSpec((_F_BLK, shp), lambda i: (i, 0))
    return pl.pallas_call(
        body,
        grid=(N // _F_BLK,),
        in_specs=[row_spec(D), row_spec(D), row_spec(1), row_spec(1),
                  pl.BlockSpec((1, G), lambda i: (0, 0)),
                  pl.BlockSpec((1, G), lambda i: (0, 0))],
        out_specs=pl.BlockSpec((G, 3 * D), lambda i: (0, 0)),
        out_shape=jax.ShapeDtypeStruct((G, 3 * D), F32),
    )(acc_a, acc_b, rden, batch2, head2, tail2)


def kernel(x, edge_index, edge_attr, batch, x_id, mask, num_nodes, num_graphs,
           node_emb_table, edge_emb_table, W0, b0, W1, b1, W2, b2):
    row = edge_index[0]
    col = edge_index[1]
    attr = edge_attr
    head = jnp.searchsorted(batch, jnp.arange(G, dtype=batch.dtype))
    head = head.astype(jnp.int32)
    tail = head + 1
    xz = jnp.zeros_like(x).at[head].set(x[head]).at[tail].set(x[tail])

    z128 = jnp.zeros((NP, D), F32)
    ones80 = jnp.ones((K, D), F32)

    def split_w(W):
        return (W[:, :D].T, W[:, D:D + 6].T,
                W[:, D + 6:2 * D + 6].T, W[:, 2 * D + 6:2 * D + 12].T,
                W[:, 2 * D + 12:].T)

    wat0, wxat0, wbt0, wxbt0, wct0 = split_w(W0)
    wat1, wxat1, wbt1, wxbt1, wct1 = split_w(W1)
    wat2, wxat2, wbt2, wxbt2, wct2 = split_w(W2)

    acc0 = _sc_pass0(edge_emb_table, col, attr, z128).reshape(NC, NP, D)[:, :N]
    dega = _sc_deg(col, ones80, z128).reshape(NC, NP, D)[:, :N]
    rden = _tc_rden(dega[0, :, 0:1], dega[1, :, 0:1])
    c0 = _tc_c0(edge_emb_table, wct0)

    p0, q0 = _tc_pq(acc0[0], acc0[1], rden, xz,
                    wat0, wxat0, wbt0, wxbt0, b0.reshape(1, D))
    ea1, acc1 = _sc_edge_first(p0, q0, c0, row, col, attr, z128)
    acc1 = acc1.reshape(NC, NP, D)[:, :N]

    p1, q1 = _tc_pq(acc1[0], acc1[1], rden, xz,
                    wat1, wxat1, wbt1, wxbt1, b1.reshape(1, D))
    m1 = _tc_m(ea1, wct1)
    ea2, acc2 = _sc_edge_mid(p1, q1, m1, row, col, attr, z128)
    acc2 = acc2.reshape(NC, NP, D)[:, :N]

    p2, q2 = _tc_pq(acc2[0], acc2[1], rden, xz,
                    wat2, wxat2, wbt2, wxbt2, b2.reshape(1, D))
    m2 = _tc_m(ea2, wct2)
    acc3 = _sc_edge_last(p2, q2, m2, row, col, attr,
                         z128).reshape(NC, NP, D)[:, :N]

    return _tc_final(acc3[0], acc3[1], rden, batch.reshape(N, 1),
                     head.reshape(1, G), tail.reshape(1, G))
